# baseline (speedup 1.0000x reference)
import functools

import jax
import jax.numpy as jnp
from jax import lax
from jax.experimental import pallas as pl
from jax.experimental.pallas import tpu as pltpu
from jax.experimental.pallas import tpu_sc as plsc

D = 32
NC = 2
NS = 16
NW = NC * NS
LANES = 16
TW = 128
DT = D // 8


@functools.cache
def _transpose_call(v: int):
    v_pad = ((v + TW - 1) // TW) * TW
    VB = 768
    n_blk = (v - (v % VB)) // VB
    assert n_blk * VB + TW == v_pad or n_blk * VB == v_pad
    n_tails = (v_pad - n_blk * VB) // TW
    iters = (n_blk + NW - 1) // NW

    mesh = plsc.VectorSubcoreMesh(core_axis_name="c", subcore_axis_name="s")

    @functools.partial(
        pl.kernel,
        mesh=mesh,
        compiler_params=pltpu.CompilerParams(
            use_tc_tiling_on_sc=True,
            needs_layout_passes=False,
            disable_bounds_checks=True,
        ),
        out_type=[
            jax.ShapeDtypeStruct((v_pad * D // 128, 128), jnp.float32)
        ] * 3,
        scratch_types=[
            *[pltpu.VMEM((8, VB), jnp.float32) for _ in range(2 * DT)],
            *[pltpu.VMEM((VB * D // 128, 128), jnp.float32) for _ in range(2)],
            *[pltpu.SemaphoreType.DMA for _ in range(4)],
        ],
    )
    def k(bT, pT, gT, b_rm, p_rm, g_rm,
          i00, i01, i02, i03, i10, i11, i12, i13,
          t0, t1, si0, si1, so0, so1):
        cid = lax.axis_index("c")
        sid = lax.axis_index("s")
        wid = sid * NC + cid
        srcs = [bT, pT, gT]
        dsts = [b_rm, p_rm, g_rm]
        ins = [[i00, i01, i02, i03], [i10, i11, i12, i13]]
        touts = [t0, t1]
        sis = [si0, si1]
        sos = [so0, so1]

        def fire_in(tbl, v0, s, w=VB):
            v0 = pl.multiple_of(v0, TW)
            for dt in range(DT):
                pltpu.async_copy(
                    srcs[tbl].at[pl.ds(dt * 8, 8), pl.ds(v0, w)],
                    ins[s][dt].at[:, pl.ds(0, w)], sis[s])

        def wait_in(tbl, v0, s, w=VB):
            v0 = pl.multiple_of(v0, TW)
            for dt in range(DT):
                pltpu.make_async_copy(
                    srcs[tbl].at[pl.ds(dt * 8, 8), pl.ds(v0, w)],
                    ins[s][dt].at[:, pl.ds(0, w)], sis[s]).wait()

        def fire_out(tbl, v0, s, w=VB):
            r0, rw = pl.multiple_of(v0 * D // 128, 8), w * D // 128
            pltpu.async_copy(
                touts[s].at[pl.ds(0, rw)], dsts[tbl].at[pl.ds(r0, rw)],
                sos[s])

        def wait_out(tbl, v0, s, w=VB):
            r0, rw = pl.multiple_of(v0 * D // 128, 8), w * D // 128
            pltpu.make_async_copy(
                touts[s].at[pl.ds(0, rw)], dsts[tbl].at[pl.ds(r0, rw)],
                sos[s]).wait()

        def transpose(s, w=VB):
            iota = lax.iota(jnp.int32, LANES)

            @plsc.parallel_loop(0, w // LANES, unroll=4)
            def _(j):
                idx_v = j * LANES + iota
                f_v = idx_v * D
                for dt in range(DT):
                    src = ins[s][dt]
                    for r0 in range(8):
                        idx_dr = (r0 + iota) & 7
                        x = plsc.load_gather(src, [idx_dr, idx_v])
                        f = f_v + (dt * 8 + idx_dr)
                        plsc.store_scatter(
                            touts[s], [f >> 7, f & 127], x)

        for tbl in range(3):
            for s in range(2):
                @pl.when(wid + NW * s < n_blk)
                def _(s=s, tbl=tbl):
                    fire_in(tbl, (wid + NW * s) * VB, s)

            def body(gg, carry, tbl=tbl):
                for s in range(2):
                    i = 2 * gg + s
                    blk = wid + NW * i

                    @pl.when(blk < n_blk)
                    def _(s=s, i=i, blk=blk):
                        wait_in(tbl, blk * VB, s)

                        @pl.when(i > 1)
                        def _():
                            wait_out(tbl, (blk - 2 * NW) * VB, s)

                        transpose(s)
                        fire_out(tbl, blk * VB, s)

                        @pl.when(blk + 2 * NW < n_blk)
                        def _():
                            fire_in(tbl, (blk + 2 * NW) * VB, s)
                return carry

            lax.fori_loop(0, (iters + 1) // 2, body, 0)
            kmax = (n_blk - 1 - wid) // NW
            for s in range(2):
                ks = kmax - ((kmax - s) % 2)

                @pl.when(ks >= 0)
                def _(s=s, ks=ks, tbl=tbl):
                    wait_out(tbl, (wid + NW * ks) * VB, s)

        if n_tails:
            def tail(c, carry):
                tv0 = (n_blk * (VB // TW) + c % n_tails) * TW
                for tbl in range(3):
                    @pl.when(wid == tbl * n_tails + c % n_tails)
                    def _(tbl=tbl):
                        fire_in(tbl, tv0, 0, TW)
                        wait_in(tbl, tv0, 0, TW)
                        transpose(0, TW)
                        fire_out(tbl, tv0, 0, TW)
                        wait_out(tbl, tv0, 0, TW)
                return carry

            lax.fori_loop(0, n_tails, tail, 0)

    return k


@functools.cache
def _sc_call(b: int, t: int, v_pad: int):
    bq = 256
    nq = b // bq
    ntask = t * nq
    per_w = ntask // NW
    seg = bq * D // DT
    assert ntask % NW == 0 and per_w % 2 == 0

    mesh = plsc.VectorSubcoreMesh(core_axis_name="c", subcore_axis_name="s")

    @functools.partial(
        pl.kernel,
        mesh=mesh,
        compiler_params=pltpu.CompilerParams(
            use_tc_tiling_on_sc=False,
            needs_layout_passes=False,
        ),
        out_type=jax.ShapeDtypeStruct((t, DT, nq * seg), jnp.float32),
        scratch_types=[
            *[pltpu.VMEM((bq,), jnp.int32) for _ in range(2)],
            *[pltpu.VMEM((bq, D), jnp.float32) for _ in range(6)],
            *[pltpu.VMEM((DT, seg), jnp.float32) for _ in range(2)],
            *[pltpu.SemaphoreType.DMA for _ in range(6)],
        ],
    )
    def k(ids_hbm, base_hbm, prior_hbm, gate_hbm, out_hbm,
          ix0, ix1, rb0, rp0, rg0, rb1, rp1, rg1, st0, st1,
          qi0, qi1, qg0, qg1, qo0, qo1):
        cid = lax.axis_index("c")
        sid = lax.axis_index("s")
        wid = sid * NC + cid
        k0 = wid * per_w
        ix = [ix0, ix1]
        rows = [(rb0, rp0, rg0), (rb1, rp1, rg1)]
        st = [st0, st1]
        qi = [qi0, qi1]
        qg = [qg0, qg1]
        qo = [qo0, qo1]
        iota = lax.iota(jnp.int32, LANES)

        def tq(c):
            kk = k0 + c
            return kk // nq, kk % nq

        def fire_idx(c, s):
            tt, qq = tq(c)
            pltpu.async_copy(
                ids_hbm.at[tt, pl.ds(qq * bq, bq)], ix[s], qi[s])

        def wait_idx(c, s):
            tt, qq = tq(c)
            pltpu.make_async_copy(
                ids_hbm.at[tt, pl.ds(qq * bq, bq)], ix[s], qi[s]).wait()

        def fire_g(c, s):
            rb, rp, rg = rows[s]
            pltpu.async_copy(base_hbm.at[ix[s]], rb, qg[s])
            pltpu.async_copy(prior_hbm.at[ix[s]], rp, qg[s])
            pltpu.async_copy(gate_hbm.at[ix[s]], rg, qg[s])

        def wait_g(c, s):
            rb, rp, rg = rows[s]
            pltpu.make_async_copy(base_hbm.at[ix[s]], rb, qg[s]).wait()
            pltpu.make_async_copy(prior_hbm.at[ix[s]], rp, qg[s]).wait()
            pltpu.make_async_copy(gate_hbm.at[ix[s]], rg, qg[s]).wait()

        def fire_out(c, s):
            tt, qq = tq(c)
            for dt in range(DT):
                pltpu.async_copy(
                    st[s].at[dt],
                    out_hbm.at[tt, dt, pl.ds(qq * seg, seg)], qo[s])

        def wait_out(c, s):
            tt, qq = tq(c)
            for dt in range(DT):
                pltpu.make_async_copy(
                    st[s].at[dt],
                    out_hbm.at[tt, dt, pl.ds(qq * seg, seg)], qo[s]).wait()

        def compute(s):
            rb, rp, rg = rows[s]

            def body(tg, carry):
                tok = tg * LANES + iota
                base_pos = ((tok >> 7) << 10) + (tok & 127)
                for d0 in range(D):
                    dv = (d0 + iota) & (D - 1)
                    gb = plsc.load_gather(rb, [tok, dv])
                    gp = plsc.load_gather(rp, [tok, dv])
                    gg = plsc.load_gather(rg, [tok, dv])
                    w = 1.0 / (1.0 + jnp.exp(-gg))
                    r = gb + w * gp
                    pos = base_pos + ((dv & 7) << 7)
                    plsc.store_scatter(st[s], [dv >> 3, pos], r)
                return carry

            lax.fori_loop(0, bq // LANES, body, 0)

        fire_idx(0, 0)
        fire_idx(1, 1)
        wait_idx(0, 0)
        fire_g(0, 0)

        def body(gg, carry):
            for s in range(2):
                c = 2 * gg + s

                @pl.when(c + 1 < per_w)
                def _(s=s, c=c):
                    wait_idx(c + 1, 1 - s)
                    fire_g(c + 1, 1 - s)

                wait_g(c, s)

                @pl.when(c + 2 < per_w)
                def _(s=s, c=c):
                    fire_idx(c + 2, s)

                @pl.when(c >= 2)
                def _(s=s, c=c):
                    wait_out(c - 2, s)

                compute(s)
                fire_out(c, s)
            return carry

        lax.fori_loop(0, per_w // 2, body, 0)
        wait_out(per_w - 2, 0)
        wait_out(per_w - 1, 1)

    return k


def kernel(input_ids, base_weight, prior_matrix, gate_logits):
    b, t = input_ids.shape
    v = base_weight.shape[0]
    v_pad = ((v + TW - 1) // TW) * TW
    ids_t = input_ids.T.astype(jnp.int32)
    bw, pm, gl = _transpose_call(v)(
        base_weight.T, prior_matrix.T, gate_logits.T)
    bw, pm, gl = (x.reshape(v_pad, D) for x in (bw, pm, gl))
    out = _sc_call(b, t, v_pad)(ids_t, bw, pm, gl)
    out5 = out.reshape(t, D // 8, b // 128, 8, 128)
    return out5.transpose((2, 4, 0, 1, 3)).reshape(b, t, D)


# speedup vs baseline: 1.1382x; 1.1382x over previous
import functools

import jax
import jax.numpy as jnp
from jax import lax
from jax.experimental import pallas as pl
from jax.experimental.pallas import tpu as pltpu
from jax.experimental.pallas import tpu_sc as plsc

D = 32
NC = 2
NS = 16
NW = NC * NS
LANES = 16
TW = 128
DT = D // 8


@functools.cache
def _transpose_call(v: int):
    v_pad = ((v + TW - 1) // TW) * TW
    VB = 768
    n_blk = (v - (v % VB)) // VB
    assert n_blk * VB + TW == v_pad or n_blk * VB == v_pad
    n_tails = (v_pad - n_blk * VB) // TW
    iters = (n_blk + NW - 1) // NW

    mesh = plsc.VectorSubcoreMesh(core_axis_name="c", subcore_axis_name="s")

    @functools.partial(
        pl.kernel,
        mesh=mesh,
        compiler_params=pltpu.CompilerParams(
            use_tc_tiling_on_sc=True,
            needs_layout_passes=False,
            disable_bounds_checks=True,
        ),
        out_type=[
            jax.ShapeDtypeStruct((v_pad * D // 128, 128), jnp.float32)
        ] * 3,
        scratch_types=[
            *[pltpu.VMEM((8, VB), jnp.float32) for _ in range(2 * DT)],
            *[pltpu.VMEM((VB * D // 128, 128), jnp.float32) for _ in range(2)],
            *[pltpu.SemaphoreType.DMA for _ in range(4)],
        ],
    )
    def k(bT, pT, gT, b_rm, p_rm, g_rm,
          i00, i01, i02, i03, i10, i11, i12, i13,
          t0, t1, si0, si1, so0, so1):
        cid = lax.axis_index("c")
        sid = lax.axis_index("s")
        wid = sid * NC + cid
        srcs = [bT, pT, gT]
        dsts = [b_rm, p_rm, g_rm]
        ins = [[i00, i01, i02, i03], [i10, i11, i12, i13]]
        touts = [t0, t1]
        sis = [si0, si1]
        sos = [so0, so1]

        def fire_in(tbl, v0, s, w=VB):
            v0 = pl.multiple_of(v0, TW)
            for dt in range(DT):
                pltpu.async_copy(
                    srcs[tbl].at[pl.ds(dt * 8, 8), pl.ds(v0, w)],
                    ins[s][dt].at[:, pl.ds(0, w)], sis[s])

        def wait_in(tbl, v0, s, w=VB):
            v0 = pl.multiple_of(v0, TW)
            for dt in range(DT):
                pltpu.make_async_copy(
                    srcs[tbl].at[pl.ds(dt * 8, 8), pl.ds(v0, w)],
                    ins[s][dt].at[:, pl.ds(0, w)], sis[s]).wait()

        def fire_out(tbl, v0, s, w=VB):
            r0, rw = pl.multiple_of(v0 * D // 128, 8), w * D // 128
            pltpu.async_copy(
                touts[s].at[pl.ds(0, rw)], dsts[tbl].at[pl.ds(r0, rw)],
                sos[s])

        def wait_out(tbl, v0, s, w=VB):
            r0, rw = pl.multiple_of(v0 * D // 128, 8), w * D // 128
            pltpu.make_async_copy(
                touts[s].at[pl.ds(0, rw)], dsts[tbl].at[pl.ds(r0, rw)],
                sos[s]).wait()

        def transpose(s, w=VB):
            iota = lax.iota(jnp.int32, LANES)

            @plsc.parallel_loop(0, w // LANES, unroll=4)
            def _(j):
                idx_v = j * LANES + iota
                f_v = idx_v * D
                for dt in range(DT):
                    src = ins[s][dt]
                    for r0 in range(8):
                        idx_dr = (r0 + iota) & 7
                        x = plsc.load_gather(src, [idx_dr, idx_v])
                        f = f_v + (dt * 8 + idx_dr)
                        plsc.store_scatter(
                            touts[s], [f >> 7, f & 127], x)

        for tbl in range(3):
            for s in range(2):
                @pl.when(wid + NW * s < n_blk)
                def _(s=s, tbl=tbl):
                    fire_in(tbl, (wid + NW * s) * VB, s)

            def body(gg, carry, tbl=tbl):
                for s in range(2):
                    i = 2 * gg + s
                    blk = wid + NW * i

                    @pl.when(blk < n_blk)
                    def _(s=s, i=i, blk=blk):
                        wait_in(tbl, blk * VB, s)

                        @pl.when(i > 1)
                        def _():
                            wait_out(tbl, (blk - 2 * NW) * VB, s)

                        transpose(s)
                        fire_out(tbl, blk * VB, s)

                        @pl.when(blk + 2 * NW < n_blk)
                        def _():
                            fire_in(tbl, (blk + 2 * NW) * VB, s)
                return carry

            lax.fori_loop(0, (iters + 1) // 2, body, 0)
            kmax = (n_blk - 1 - wid) // NW
            for s in range(2):
                ks = kmax - ((kmax - s) % 2)

                @pl.when(ks >= 0)
                def _(s=s, ks=ks, tbl=tbl):
                    wait_out(tbl, (wid + NW * ks) * VB, s)

        if n_tails:
            def tail(c, carry):
                tv0 = (n_blk * (VB // TW) + c % n_tails) * TW
                for tbl in range(3):
                    @pl.when(wid == tbl * n_tails + c % n_tails)
                    def _(tbl=tbl):
                        fire_in(tbl, tv0, 0, TW)
                        wait_in(tbl, tv0, 0, TW)
                        transpose(0, TW)
                        fire_out(tbl, tv0, 0, TW)
                        wait_out(tbl, tv0, 0, TW)
                return carry

            lax.fori_loop(0, n_tails, tail, 0)

    return k


@functools.cache
def _sc_call(b: int, t: int, v_pad: int):
    bq = 256
    nq = b // bq
    ntask = t * nq
    per_w = ntask // NW
    seg = bq * D // DT
    assert ntask % NW == 0 and per_w % 2 == 0

    mesh = plsc.VectorSubcoreMesh(core_axis_name="c", subcore_axis_name="s")

    @functools.partial(
        pl.kernel,
        mesh=mesh,
        compiler_params=pltpu.CompilerParams(
            use_tc_tiling_on_sc=False,
            needs_layout_passes=False,
        ),
        out_type=jax.ShapeDtypeStruct((t, DT, nq * seg), jnp.float32),
        scratch_types=[
            *[pltpu.VMEM((bq,), jnp.int32) for _ in range(2)],
            *[pltpu.VMEM((bq, D), jnp.float32) for _ in range(6)],
            *[pltpu.VMEM((DT, seg), jnp.float32) for _ in range(2)],
            *[pltpu.SemaphoreType.DMA for _ in range(6)],
        ],
    )
    def k(ids_hbm, base_hbm, prior_hbm, gate_hbm, out_hbm,
          ix0, ix1, rb0, rp0, rg0, rb1, rp1, rg1, st0, st1,
          qi0, qi1, qg0, qg1, qo0, qo1):
        cid = lax.axis_index("c")
        sid = lax.axis_index("s")
        wid = sid * NC + cid
        k0 = wid * per_w
        ix = [ix0, ix1]
        rows = [(rb0, rp0, rg0), (rb1, rp1, rg1)]
        st = [st0, st1]
        qi = [qi0, qi1]
        qg = [qg0, qg1]
        qo = [qo0, qo1]
        iota = lax.iota(jnp.int32, LANES)

        def tq(c):
            kk = k0 + c
            return kk // nq, kk % nq

        def fire_idx(c, s):
            tt, qq = tq(c)
            pltpu.async_copy(
                ids_hbm.at[tt, pl.ds(qq * bq, bq)], ix[s], qi[s])

        def wait_idx(c, s):
            tt, qq = tq(c)
            pltpu.make_async_copy(
                ids_hbm.at[tt, pl.ds(qq * bq, bq)], ix[s], qi[s]).wait()

        def fire_g(c, s):
            rb, rp, rg = rows[s]
            pltpu.async_copy(base_hbm.at[ix[s]], rb, qg[s])
            pltpu.async_copy(prior_hbm.at[ix[s]], rp, qg[s])
            pltpu.async_copy(gate_hbm.at[ix[s]], rg, qg[s])

        def wait_g(c, s):
            rb, rp, rg = rows[s]
            pltpu.make_async_copy(base_hbm.at[ix[s]], rb, qg[s]).wait()
            pltpu.make_async_copy(prior_hbm.at[ix[s]], rp, qg[s]).wait()
            pltpu.make_async_copy(gate_hbm.at[ix[s]], rg, qg[s]).wait()

        def fire_out(c, s):
            tt, qq = tq(c)
            for dt in range(DT):
                pltpu.async_copy(
                    st[s].at[dt],
                    out_hbm.at[tt, dt, pl.ds(qq * seg, seg)], qo[s])

        def wait_out(c, s):
            tt, qq = tq(c)
            for dt in range(DT):
                pltpu.make_async_copy(
                    st[s].at[dt],
                    out_hbm.at[tt, dt, pl.ds(qq * seg, seg)], qo[s]).wait()

        def compute(s):
            rb, rp, rg = rows[s]

            @plsc.parallel_loop(0, bq // LANES, unroll=2)
            def body(tg):
                tok = tg * LANES + iota
                base_pos = ((tok >> 7) << 10) + (tok & 127)
                for d0 in range(D):
                    dv = (d0 + iota) & (D - 1)
                    gb = plsc.load_gather(rb, [tok, dv])
                    gp = plsc.load_gather(rp, [tok, dv])
                    gg = plsc.load_gather(rg, [tok, dv])
                    w = 1.0 / (1.0 + jnp.exp(-gg))
                    r = gb + w * gp
                    pos = base_pos + ((dv & 7) << 7)
                    plsc.store_scatter(st[s], [dv >> 3, pos], r)

        fire_idx(0, 0)
        fire_idx(1, 1)
        wait_idx(0, 0)
        fire_g(0, 0)

        def body(gg, carry):
            for s in range(2):
                c = 2 * gg + s

                @pl.when(c + 1 < per_w)
                def _(s=s, c=c):
                    wait_idx(c + 1, 1 - s)
                    fire_g(c + 1, 1 - s)

                wait_g(c, s)

                @pl.when(c + 2 < per_w)
                def _(s=s, c=c):
                    fire_idx(c + 2, s)

                @pl.when(c >= 2)
                def _(s=s, c=c):
                    wait_out(c - 2, s)

                compute(s)
                fire_out(c, s)
            return carry

        lax.fori_loop(0, per_w // 2, body, 0)
        wait_out(per_w - 2, 0)
        wait_out(per_w - 1, 1)

    return k


def kernel(input_ids, base_weight, prior_matrix, gate_logits):
    b, t = input_ids.shape
    v = base_weight.shape[0]
    v_pad = ((v + TW - 1) // TW) * TW
    ids_t = input_ids.T.astype(jnp.int32)
    bw, pm, gl = _transpose_call(v)(
        base_weight.T, prior_matrix.T, gate_logits.T)
    bw, pm, gl = (x.reshape(v_pad, D) for x in (bw, pm, gl))
    out = _sc_call(b, t, v_pad)(ids_t, bw, pm, gl)
    out5 = out.reshape(t, D // 8, b // 128, 8, 128)
    return out5.transpose((2, 4, 0, 1, 3)).reshape(b, t, D)


# speedup vs baseline: 1.6560x; 1.4549x over previous
import functools

import jax
import jax.numpy as jnp
from jax import lax
from jax.experimental import pallas as pl
from jax.experimental.pallas import tpu as pltpu
from jax.experimental.pallas import tpu_sc as plsc

D = 32
NC = 2
NS = 16
NW = NC * NS
LANES = 16
TW = 128


@functools.cache
def _transpose_call(v: int):
    v_pad = ((v + TW - 1) // TW) * TW
    VB = 768
    n_blk = (v - (v % VB)) // VB
    assert n_blk * VB + TW == v_pad or n_blk * VB == v_pad
    n_tails = (v_pad - n_blk * VB) // TW
    iters = (n_blk + NW - 1) // NW
    DT = D // 8

    mesh = plsc.VectorSubcoreMesh(core_axis_name="c", subcore_axis_name="s")

    @functools.partial(
        pl.kernel,
        mesh=mesh,
        compiler_params=pltpu.CompilerParams(
            use_tc_tiling_on_sc=True,
            needs_layout_passes=False,
            disable_bounds_checks=True,
        ),
        out_type=[
            jax.ShapeDtypeStruct((v_pad * D // 128, 128), jnp.float32)
        ] * 3,
        scratch_types=[
            *[pltpu.VMEM((8, VB), jnp.float32) for _ in range(2 * DT)],
            *[pltpu.VMEM((VB * D // 128, 128), jnp.float32) for _ in range(2)],
            *[pltpu.SemaphoreType.DMA for _ in range(4)],
        ],
    )
    def k(bT, pT, gT, b_rm, p_rm, g_rm,
          i00, i01, i02, i03, i10, i11, i12, i13,
          t0, t1, si0, si1, so0, so1):
        cid = lax.axis_index("c")
        sid = lax.axis_index("s")
        wid = sid * NC + cid
        srcs = [bT, pT, gT]
        dsts = [b_rm, p_rm, g_rm]
        ins = [[i00, i01, i02, i03], [i10, i11, i12, i13]]
        touts = [t0, t1]
        sis = [si0, si1]
        sos = [so0, so1]

        def fire_in(tbl, v0, s, w=VB):
            v0 = pl.multiple_of(v0, TW)
            for dt in range(DT):
                pltpu.async_copy(
                    srcs[tbl].at[pl.ds(dt * 8, 8), pl.ds(v0, w)],
                    ins[s][dt].at[:, pl.ds(0, w)], sis[s])

        def wait_in(tbl, v0, s, w=VB):
            v0 = pl.multiple_of(v0, TW)
            for dt in range(DT):
                pltpu.make_async_copy(
                    srcs[tbl].at[pl.ds(dt * 8, 8), pl.ds(v0, w)],
                    ins[s][dt].at[:, pl.ds(0, w)], sis[s]).wait()

        def fire_out(tbl, v0, s, w=VB):
            r0, rw = pl.multiple_of(v0 * D // 128, 8), w * D // 128
            pltpu.async_copy(
                touts[s].at[pl.ds(0, rw)], dsts[tbl].at[pl.ds(r0, rw)],
                sos[s])

        def wait_out(tbl, v0, s, w=VB):
            r0, rw = pl.multiple_of(v0 * D // 128, 8), w * D // 128
            pltpu.make_async_copy(
                touts[s].at[pl.ds(0, rw)], dsts[tbl].at[pl.ds(r0, rw)],
                sos[s]).wait()

        def transpose(s, w=VB):
            iota = lax.iota(jnp.int32, LANES)

            @plsc.parallel_loop(0, w // LANES, unroll=4)
            def _(j):
                idx_v = j * LANES + iota
                f_v = idx_v * D
                for dt in range(DT):
                    src = ins[s][dt]
                    for r0 in range(8):
                        idx_dr = (r0 + iota) & 7
                        x = plsc.load_gather(src, [idx_dr, idx_v])
                        f = f_v + (dt * 8 + idx_dr)
                        plsc.store_scatter(
                            touts[s], [f >> 7, f & 127], x)

        for tbl in range(3):
            for s in range(2):
                @pl.when(wid + NW * s < n_blk)
                def _(s=s, tbl=tbl):
                    fire_in(tbl, (wid + NW * s) * VB, s)

            def body(gg, carry, tbl=tbl):
                for s in range(2):
                    i = 2 * gg + s
                    blk = wid + NW * i

                    @pl.when(blk < n_blk)
                    def _(s=s, i=i, blk=blk):
                        wait_in(tbl, blk * VB, s)

                        @pl.when(i > 1)
                        def _():
                            wait_out(tbl, (blk - 2 * NW) * VB, s)

                        transpose(s)
                        fire_out(tbl, blk * VB, s)

                        @pl.when(blk + 2 * NW < n_blk)
                        def _():
                            fire_in(tbl, (blk + 2 * NW) * VB, s)
                return carry

            lax.fori_loop(0, (iters + 1) // 2, body, 0)
            kmax = (n_blk - 1 - wid) // NW
            for s in range(2):
                ks = kmax - ((kmax - s) % 2)

                @pl.when(ks >= 0)
                def _(s=s, ks=ks, tbl=tbl):
                    wait_out(tbl, (wid + NW * ks) * VB, s)

        if n_tails:
            def tail(c, carry):
                tv0 = (n_blk * (VB // TW) + c % n_tails) * TW
                for tbl in range(3):
                    @pl.when(wid == tbl * n_tails + c % n_tails)
                    def _(tbl=tbl):
                        fire_in(tbl, tv0, 0, TW)
                        wait_in(tbl, tv0, 0, TW)
                        transpose(0, TW)
                        fire_out(tbl, tv0, 0, TW)
                        wait_out(tbl, tv0, 0, TW)
                return carry

            lax.fori_loop(0, n_tails, tail, 0)

    return k


@functools.cache
def _sc_call(b: int, t: int, v_pad: int):
    pw = (b * t) // NW
    nch = b // NW
    n = t
    assert nch % 2 == 0 and n % 8 == 0

    mesh = plsc.VectorSubcoreMesh(core_axis_name="c", subcore_axis_name="s")

    @functools.partial(
        pl.kernel,
        mesh=mesh,
        compiler_params=pltpu.CompilerParams(use_tc_tiling_on_sc=False),
        out_type=jax.ShapeDtypeStruct((b, t, D), jnp.float32),
        scratch_types=[
            pltpu.VMEM((pw,), jnp.int32),
            *[pltpu.VMEM((n, D), jnp.float32) for _ in range(8)],
            *[pltpu.SemaphoreType.DMA for _ in range(8)],
        ],
    )
    def k(ids_hbm, base_hbm, prior_hbm, gate_hbm, out_hbm,
          idx_v, b0, p0, g0, o0, b1, p1, g1, o1,
          sb0, sp0, sg0, so0, sb1, sp1, sg1, so1):
        cid = lax.axis_index("c")
        sid = lax.axis_index("s")
        wid = sid * NC + cid
        row0 = wid * nch

        pltpu.sync_copy(ids_hbm.at[wid], idx_v)

        bufs = [
            (b0, p0, g0, o0, sb0, sp0, sg0, so0),
            (b1, p1, g1, o1, sb1, sp1, sg1, so1),
        ]

        def fire_gathers(g, s):
            bb, pp, gt, _, sb, sp, sg, _ = bufs[s]
            idx = idx_v.at[pl.ds(g * n, n)]
            pltpu.async_copy(base_hbm.at[idx], bb, sb)
            pltpu.async_copy(prior_hbm.at[idx], pp, sp)
            pltpu.async_copy(gate_hbm.at[idx], gt, sg)

        def wait_gathers(g, s):
            bb, pp, gt, _, sb, sp, sg, _ = bufs[s]
            idx = idx_v.at[pl.ds(g * n, n)]
            pltpu.make_async_copy(base_hbm.at[idx], bb, sb).wait()
            pltpu.make_async_copy(prior_hbm.at[idx], pp, sp).wait()
            pltpu.make_async_copy(gate_hbm.at[idx], gt, sg).wait()

        def out_slice(g):
            return out_hbm.at[row0 + g]

        def wait_out(g, s):
            o, so = bufs[s][3], bufs[s][7]
            pltpu.make_async_copy(o, out_slice(g), so).wait()

        fire_gathers(0, 0)
        fire_gathers(1, 1)

        def body(gg, carry):
            for s in range(2):
                g = 2 * gg + s
                bb, pp, gt, o, sb, sp, sg, so = bufs[s]
                wait_gathers(g, s)

                @pl.when(gg > 0)
                def _():
                    wait_out(g - 2, s)

                def rows(i, c):
                    for r in range(2):
                        for h in range(2):
                            sl = pl.ds(h * LANES, LANES)
                            gv = gt[2 * i + r, sl]
                            w = 1.0 / (1.0 + jnp.exp(-gv))
                            o[2 * i + r, sl] = bb[2 * i + r, sl] + w * pp[2 * i + r, sl]
                    return c

                lax.fori_loop(0, n // 2, rows, 0)
                pltpu.async_copy(o, out_slice(g), so)

                @pl.when(gg < (nch // 2 - 1))
                def _():
                    fire_gathers(g + 2, s)
            return carry

        lax.fori_loop(0, nch // 2, body, 0)
        wait_out(nch - 2, 0)
        wait_out(nch - 1, 1)

    return k


def kernel(input_ids, base_weight, prior_matrix, gate_logits):
    b, t = input_ids.shape
    v = base_weight.shape[0]
    v_pad = ((v + TW - 1) // TW) * TW
    ids = input_ids.reshape(NW, (b * t) // NW).astype(jnp.int32)
    bw, pm, gl = _transpose_call(v)(
        base_weight.T, prior_matrix.T, gate_logits.T)
    bw, pm, gl = (x.reshape(v_pad, D) for x in (bw, pm, gl))
    return _sc_call(b, t, v_pad)(ids, bw, pm, gl)


# speedup vs baseline: 1.6625x; 1.0040x over previous
import functools

import jax
import jax.numpy as jnp
from jax import lax
from jax.experimental import pallas as pl
from jax.experimental.pallas import tpu as pltpu
from jax.experimental.pallas import tpu_sc as plsc

D = 32
NC = 2
NS = 16
NW = NC * NS
LANES = 16
TW = 128


@functools.cache
def _transpose_call(v: int):
    v_pad = ((v + TW - 1) // TW) * TW
    VB = 896
    n_blk = (v - (v % VB)) // VB
    assert n_blk * VB + TW == v_pad or n_blk * VB == v_pad
    n_tails = (v_pad - n_blk * VB) // TW
    iters = (n_blk + NW - 1) // NW
    DT = D // 8

    mesh = plsc.VectorSubcoreMesh(core_axis_name="c", subcore_axis_name="s")

    @functools.partial(
        pl.kernel,
        mesh=mesh,
        compiler_params=pltpu.CompilerParams(
            use_tc_tiling_on_sc=True,
            needs_layout_passes=False,
            disable_bounds_checks=True,
        ),
        out_type=[
            jax.ShapeDtypeStruct((v_pad * D // 128, 128), jnp.float32)
        ] * 3,
        scratch_types=[
            *[pltpu.VMEM((8, VB), jnp.float32) for _ in range(2 * DT)],
            *[pltpu.VMEM((VB * D // 128, 128), jnp.float32) for _ in range(2)],
            *[pltpu.SemaphoreType.DMA for _ in range(4)],
        ],
    )
    def k(bT, pT, gT, b_rm, p_rm, g_rm,
          i00, i01, i02, i03, i10, i11, i12, i13,
          t0, t1, si0, si1, so0, so1):
        cid = lax.axis_index("c")
        sid = lax.axis_index("s")
        wid = sid * NC + cid
        srcs = [bT, pT, gT]
        dsts = [b_rm, p_rm, g_rm]
        ins = [[i00, i01, i02, i03], [i10, i11, i12, i13]]
        touts = [t0, t1]
        sis = [si0, si1]
        sos = [so0, so1]

        def fire_in(tbl, v0, s, w=VB):
            v0 = pl.multiple_of(v0, TW)
            for dt in range(DT):
                pltpu.async_copy(
                    srcs[tbl].at[pl.ds(dt * 8, 8), pl.ds(v0, w)],
                    ins[s][dt].at[:, pl.ds(0, w)], sis[s])

        def wait_in(tbl, v0, s, w=VB):
            v0 = pl.multiple_of(v0, TW)
            for dt in range(DT):
                pltpu.make_async_copy(
                    srcs[tbl].at[pl.ds(dt * 8, 8), pl.ds(v0, w)],
                    ins[s][dt].at[:, pl.ds(0, w)], sis[s]).wait()

        def fire_out(tbl, v0, s, w=VB):
            r0, rw = pl.multiple_of(v0 * D // 128, 8), w * D // 128
            pltpu.async_copy(
                touts[s].at[pl.ds(0, rw)], dsts[tbl].at[pl.ds(r0, rw)],
                sos[s])

        def wait_out(tbl, v0, s, w=VB):
            r0, rw = pl.multiple_of(v0 * D // 128, 8), w * D // 128
            pltpu.make_async_copy(
                touts[s].at[pl.ds(0, rw)], dsts[tbl].at[pl.ds(r0, rw)],
                sos[s]).wait()

        def transpose(s, w=VB):
            iota = lax.iota(jnp.int32, LANES)

            @plsc.parallel_loop(0, w // LANES, unroll=4)
            def _(j):
                idx_v = j * LANES + iota
                f_v = idx_v * D
                for dt in range(DT):
                    src = ins[s][dt]
                    for r0 in range(8):
                        idx_dr = (r0 + iota) & 7
                        x = plsc.load_gather(src, [idx_dr, idx_v])
                        f = f_v + (dt * 8 + idx_dr)
                        plsc.store_scatter(
                            touts[s], [f >> 7, f & 127], x)

        for tbl in range(3):
            for s in range(2):
                @pl.when(wid + NW * s < n_blk)
                def _(s=s, tbl=tbl):
                    fire_in(tbl, (wid + NW * s) * VB, s)

            def body(gg, carry, tbl=tbl):
                for s in range(2):
                    i = 2 * gg + s
                    blk = wid + NW * i

                    @pl.when(blk < n_blk)
                    def _(s=s, i=i, blk=blk):
                        wait_in(tbl, blk * VB, s)

                        @pl.when(i > 1)
                        def _():
                            wait_out(tbl, (blk - 2 * NW) * VB, s)

                        transpose(s)
                        fire_out(tbl, blk * VB, s)

                        @pl.when(blk + 2 * NW < n_blk)
                        def _():
                            fire_in(tbl, (blk + 2 * NW) * VB, s)
                return carry

            lax.fori_loop(0, (iters + 1) // 2, body, 0)
            kmax = (n_blk - 1 - wid) // NW
            for s in range(2):
                ks = kmax - ((kmax - s) % 2)

                @pl.when(ks >= 0)
                def _(s=s, ks=ks, tbl=tbl):
                    wait_out(tbl, (wid + NW * ks) * VB, s)

        if n_tails:
            def tail(c, carry):
                tv0 = (n_blk * (VB // TW) + c % n_tails) * TW
                for tbl in range(3):
                    @pl.when(wid == tbl * n_tails + c % n_tails)
                    def _(tbl=tbl):
                        fire_in(tbl, tv0, 0, TW)
                        wait_in(tbl, tv0, 0, TW)
                        transpose(0, TW)
                        fire_out(tbl, tv0, 0, TW)
                        wait_out(tbl, tv0, 0, TW)
                return carry

            lax.fori_loop(0, n_tails, tail, 0)

    return k


@functools.cache
def _sc_call(b: int, t: int, v_pad: int):
    pw = (b * t) // NW
    nch = b // NW
    n = t
    assert nch % 2 == 0 and n % 8 == 0

    mesh = plsc.VectorSubcoreMesh(core_axis_name="c", subcore_axis_name="s")

    @functools.partial(
        pl.kernel,
        mesh=mesh,
        compiler_params=pltpu.CompilerParams(use_tc_tiling_on_sc=False),
        out_type=jax.ShapeDtypeStruct((b, t, D), jnp.float32),
        scratch_types=[
            pltpu.VMEM((pw,), jnp.int32),
            *[pltpu.VMEM((n, D), jnp.float32) for _ in range(8)],
            *[pltpu.SemaphoreType.DMA for _ in range(8)],
        ],
    )
    def k(ids_hbm, base_hbm, prior_hbm, gate_hbm, out_hbm,
          idx_v, b0, p0, g0, o0, b1, p1, g1, o1,
          sb0, sp0, sg0, so0, sb1, sp1, sg1, so1):
        cid = lax.axis_index("c")
        sid = lax.axis_index("s")
        wid = sid * NC + cid
        row0 = wid * nch

        pltpu.sync_copy(ids_hbm.at[wid], idx_v)

        bufs = [
            (b0, p0, g0, o0, sb0, sp0, sg0, so0),
            (b1, p1, g1, o1, sb1, sp1, sg1, so1),
        ]

        def fire_gathers(g, s):
            bb, pp, gt, _, sb, sp, sg, _ = bufs[s]
            idx = idx_v.at[pl.ds(g * n, n)]
            pltpu.async_copy(base_hbm.at[idx], bb, sb)
            pltpu.async_copy(prior_hbm.at[idx], pp, sp)
            pltpu.async_copy(gate_hbm.at[idx], gt, sg)

        def wait_gathers(g, s):
            bb, pp, gt, _, sb, sp, sg, _ = bufs[s]
            idx = idx_v.at[pl.ds(g * n, n)]
            pltpu.make_async_copy(base_hbm.at[idx], bb, sb).wait()
            pltpu.make_async_copy(prior_hbm.at[idx], pp, sp).wait()
            pltpu.make_async_copy(gate_hbm.at[idx], gt, sg).wait()

        def out_slice(g):
            return out_hbm.at[row0 + g]

        def wait_out(g, s):
            o, so = bufs[s][3], bufs[s][7]
            pltpu.make_async_copy(o, out_slice(g), so).wait()

        fire_gathers(0, 0)
        fire_gathers(1, 1)

        def body(gg, carry):
            for s in range(2):
                g = 2 * gg + s
                bb, pp, gt, o, sb, sp, sg, so = bufs[s]
                wait_gathers(g, s)

                @pl.when(gg > 0)
                def _():
                    wait_out(g - 2, s)

                def rows(i, c):
                    for r in range(2):
                        for h in range(2):
                            sl = pl.ds(h * LANES, LANES)
                            gv = gt[2 * i + r, sl]
                            w = 1.0 / (1.0 + jnp.exp(-gv))
                            o[2 * i + r, sl] = bb[2 * i + r, sl] + w * pp[2 * i + r, sl]
                    return c

                lax.fori_loop(0, n // 2, rows, 0)
                pltpu.async_copy(o, out_slice(g), so)

                @pl.when(gg < (nch // 2 - 1))
                def _():
                    fire_gathers(g + 2, s)
            return carry

        lax.fori_loop(0, nch // 2, body, 0)
        wait_out(nch - 2, 0)
        wait_out(nch - 1, 1)

    return k


def kernel(input_ids, base_weight, prior_matrix, gate_logits):
    b, t = input_ids.shape
    v = base_weight.shape[0]
    v_pad = ((v + TW - 1) // TW) * TW
    ids = input_ids.reshape(NW, (b * t) // NW).astype(jnp.int32)
    bw, pm, gl = _transpose_call(v)(
        base_weight.T, prior_matrix.T, gate_logits.T)
    bw, pm, gl = (x.reshape(v_pad, D) for x in (bw, pm, gl))
    return _sc_call(b, t, v_pad)(ids, bw, pm, gl)
